# Initial kernel scaffold; baseline (speedup 1.0000x reference)
import jax
import jax.numpy as jnp
from jax.experimental import pallas as pl


def kernel(context_vec, stem_k3, stem_w1, stem_bias, b0_ln, b0_w4, b0_w1, b0_bias, b1_ln, b1_w4, b1_w1, b1_bias, b2_ln, b2_w4, b2_w1, b2_bias, b3_ln, b3_w4, b3_w1, b3_bias, b3_wh):
    raise NotImplementedError("write your pallas kernel here")


# speedup vs baseline: 1.2916x; 1.2916x over previous
import functools

import jax
import jax.numpy as jnp
from jax import lax
from jax.experimental import pallas as pl
from jax.experimental.pallas import tpu as pltpu

_BF16 = jnp.bfloat16
_F32 = jnp.float32


def _sigmoid(x):
    return 1.0 / (1.0 + jnp.exp(-x))


def _silu(x):
    return x * _sigmoid(x)


def _phase_pair(V, Vm, Vp, wa0, wa1, wb0, wb1, w1, bias, wh, has_head):
    outs = []
    for wa, wb, Vs in ((wa0, wb0, Vm), (wa1, wb1, Vp)):
        acc = jnp.dot(wa, V, preferred_element_type=_F32)
        acc = acc + jnp.dot(wb, Vs, preferred_element_type=_F32)
        acc = acc + bias[:, 0:1]
        a = _silu(acc)
        h = jnp.dot(w1, a.astype(_BF16), preferred_element_type=_F32)
        h = _silu(h + bias[:, 1:2])
        if has_head:
            o = jnp.dot(wh, h.astype(_BF16), preferred_element_type=_F32)
            o = _sigmoid(o + bias[0:3, 2:3])
            outs.append(o)
        else:
            outs.append(h)
    return outs


def _col_shifts(V, W, HW):
    lane = lax.broadcasted_iota(jnp.int32, (1, HW), 1)
    col = lane % W
    z1 = jnp.zeros((V.shape[0], 1), _BF16)
    Vm = jnp.concatenate([z1, V[:, : HW - 1]], axis=1)
    Vm = jnp.where(col > 0, Vm, jnp.asarray(0, _BF16))
    Vp = jnp.concatenate([V[:, 1:], z1], axis=1)
    Vp = jnp.where(col < W - 1, Vp, jnp.asarray(0, _BF16))
    return Vm, Vp


def _ln_normalize(x, g, b):
    n = x.size
    s = jnp.sum(x)
    s2 = jnp.sum(x * x)
    mu = s / n
    var = s2 / n - mu * mu
    xn = (x - mu) * lax.rsqrt(var + 1e-5)
    return (xn * g + b).astype(_BF16)


def _row_shift_m(xh, C, W, HW):
    zW = jnp.zeros((C, W), _BF16)
    return jnp.concatenate([zW, xh[:, : HW - W]], axis=1)


def _row_shift_p(xh, C, W, HW):
    zW = jnp.zeros((C, W), _BF16)
    return jnp.concatenate([xh[:, W:], zW], axis=1)


def _stem_b0_kernel(x_ref, k9_ref, w1s_ref, sb_ref, ln_ref, wa_ref, wb_ref,
                    w1_ref, bias_ref, o_ref, *, H, W, C1):
    HW = H * W
    x = x_ref[...]
    lane = lax.broadcasted_iota(jnp.int32, (1, HW), 1)
    col = lane % W
    row = lane // W

    rows = []
    for dy in (-1, 0, 1):
        for dx in (-1, 0, 1):
            s = dy * W + dx
            sh = x if s == 0 else pltpu.roll(x, (-s) % HW, axis=1)
            ok = ((col + dx >= 0) & (col + dx < W) &
                  (row + dy >= 0) & (row + dy < H))
            rows.append(jnp.where(ok, sh, 0.0))
    S = jnp.concatenate(rows, axis=0).astype(_BF16)
    y = jnp.dot(k9_ref[...], S, preferred_element_type=_F32)
    y = y + sb_ref[:C1, 0:1]
    h = jnp.dot(w1s_ref[...], y.astype(_BF16), preferred_element_type=_F32)
    h = _silu(h + sb_ref[:, 1:2])

    C = h.shape[0]
    xh = _ln_normalize(h, ln_ref[0], ln_ref[1])
    for py in (0, 1):
        sy = _row_shift_m(xh, C, W, HW) if py == 0 else _row_shift_p(xh, C, W, HW)
        V = jnp.concatenate([xh, sy], axis=0)
        Vm, Vp = _col_shifts(V, W, HW)
        outs = _phase_pair(V, Vm, Vp,
                           wa_ref[py, 0], wa_ref[py, 1],
                           wb_ref[py, 0], wb_ref[py, 1],
                           w1_ref[...], bias_ref[...], None, False)
        o_ref[py, 0] = outs[0]
        o_ref[py, 1] = outs[1]


def _run_stem_b0(x2d, k9, w1s, sb, ln, wa, wb, w1, bias, H, W):
    C1 = k9.shape[0]
    c_out = w1.shape[0]
    HW = H * W
    args = (x2d, k9, w1s, sb, ln, wa, wb, w1, bias)
    whole = lambda a: pl.BlockSpec(a.shape, lambda _n=a.ndim: (0,) * _n)
    return pl.pallas_call(
        functools.partial(_stem_b0_kernel, H=H, W=W, C1=C1),
        out_shape=jax.ShapeDtypeStruct((2, 2, c_out, HW), _F32),
        grid=(),
        in_specs=[whole(a) for a in args],
        out_specs=pl.BlockSpec((2, 2, c_out, HW), lambda: (0, 0, 0, 0)),
    )(*args)


def _up_kernel(x_ref, ln_ref, wa_ref, wb_ref, w1_ref, bias_ref, *rest,
               W, has_head):
    if has_head:
        wh_ref, o_ref = rest
        wh = wh_ref[...]
    else:
        (o_ref,) = rest
        wh = None
    x = x_ref[...]
    C, HW = x.shape
    py = pl.program_id(0)

    xh = _ln_normalize(x, ln_ref[0], ln_ref[1])
    s_ym = _row_shift_m(xh, C, W, HW)
    s_yp = _row_shift_p(xh, C, W, HW)
    sy = jnp.where(py == 0, s_ym, s_yp)
    V = jnp.concatenate([xh, sy], axis=0)
    Vm, Vp = _col_shifts(V, W, HW)
    outs = _phase_pair(V, Vm, Vp,
                       wa_ref[0, 0], wa_ref[0, 1],
                       wb_ref[0, 0], wb_ref[0, 1],
                       w1_ref[...], bias_ref[...], wh, has_head)
    o_ref[0, 0] = outs[0]
    o_ref[0, 1] = outs[1]


def _run_up(x, ln, wa, wb, w1, bias, wh, H, W, has_head):
    C_in, HW = x.shape
    C_out = w1.shape[0]
    C_res = 3 if has_head else C_out
    args = [x, ln, wa, wb, w1, bias]
    if has_head:
        args.append(wh)

    def const_spec(a):
        return pl.BlockSpec(a.shape, lambda py, _n=a.ndim: (0,) * _n)

    in_specs = [const_spec(x), const_spec(ln),
                pl.BlockSpec((1, 2) + wa.shape[2:], lambda py: (py, 0, 0, 0)),
                pl.BlockSpec((1, 2) + wb.shape[2:], lambda py: (py, 0, 0, 0)),
                const_spec(w1), const_spec(bias)]
    if has_head:
        in_specs.append(const_spec(wh))
    return pl.pallas_call(
        functools.partial(_up_kernel, W=W, has_head=has_head),
        out_shape=jax.ShapeDtypeStruct((2, 2, C_res, HW), _F32),
        grid=(2,),
        in_specs=in_specs,
        out_specs=pl.BlockSpec((1, 2, C_res, HW), lambda py: (py, 0, 0, 0)),
        compiler_params=pltpu.CompilerParams(
            dimension_semantics=("parallel",)),
    )(*args)


def _mix(phases, H, W):
    C = phases.shape[2]
    o = phases.reshape(2, 2, C, H, W)
    o = o.transpose(2, 3, 0, 4, 1)
    return o.reshape(C, 4 * H * W)


def _prep_taps(w4):
    C_out, C_in = w4.shape[1], w4.shape[2]
    w4r = w4.reshape(2, 2, 4, C_out, C_in)
    wa = jnp.concatenate([w4r[:, :, 0], w4r[:, :, 2]], axis=-1)
    wb = jnp.concatenate([w4r[:, :, 1], w4r[:, :, 3]], axis=-1)
    return wa.astype(_BF16), wb.astype(_BF16)


def kernel(context_vec, stem_k3, stem_w1, stem_bias,
           b0_ln, b0_w4, b0_w1, b0_bias,
           b1_ln, b1_w4, b1_w1, b1_bias,
           b2_ln, b2_w4, b2_w1, b2_bias,
           b3_ln, b3_w4, b3_w1, b3_bias, b3_wh):
    H, W = context_vec.shape[-2], context_vec.shape[-1]
    x2d = context_vec.reshape(1, H * W).astype(_F32)

    k9 = stem_k3.astype(_BF16)
    w1s = stem_w1.astype(_BF16)
    wa0, wb0 = _prep_taps(b0_w4)
    phases = _run_stem_b0(x2d, k9, w1s, stem_bias, b0_ln, wa0, wb0,
                          b0_w1.astype(_BF16), b0_bias, H, W)

    blocks = [
        (b1_ln, b1_w4, b1_w1, b1_bias, None, False),
        (b2_ln, b2_w4, b2_w1, b2_bias, None, False),
        (b3_ln, b3_w4, b3_w1, b3_bias, b3_wh, True),
    ]
    for ln, w4, w1, bias, wh, has_head in blocks:
        h = _mix(phases, H, W)
        H, W = 2 * H, 2 * W
        wa, wb = _prep_taps(w4)
        whb = wh.astype(_BF16) if wh is not None else None
        phases = _run_up(h, ln, wa, wb, w1.astype(_BF16), bias, whb,
                         H, W, has_head)

    img = _mix(phases, H, W)
    return img.reshape(1, 3, 2 * H, 2 * W)
